# baseline (speedup 1.0000x reference)
import jax
import jax.numpy as jnp
from jax import lax
from jax.experimental import pallas as pl
from jax.experimental.pallas import tpu as pltpu
from jax.experimental.pallas import tpu_sc as plsc

_BATCH = 32768
_NIN = 64
_NOUT = 256
_NC = 2
_NS = 16
_NW = _NC * _NS
_ROWS_PER_W = _BATCH // _NW
_CHUNK = 64
_NCHUNKS = _ROWS_PER_W // _CHUNK
_L = 16
_RG = _CHUNK // _L
_UJ = 2
_TI = 8
_TC = 128
_TJW = _ROWS_PER_W // _TC
_NPAD = _NOUT + 8


def _sc_body(q_hbm, w_hbm, idx_hbm, out_hbm,
             x_v, pad_v, out_v0, out_v1, w_v, idx_v, wexp_v, cexp_v,
             sem_x, sem_o0, sem_o1):
    wid = lax.axis_index("s") * _NC + lax.axis_index("c")
    tj0 = wid * _TJW
    base_row = wid * _ROWS_PER_W

    out_bufs = (out_v0, out_v1)
    o_sems = (sem_o0, sem_o1)

    x_dmas = [
        pltpu.async_copy(
            q_hbm.at[pl.ds(ti * (_BATCH // _TC) + tj0, _TJW)],
            x_v.at[pl.ds(ti * _TJW, _TJW)],
            sem_x)
        for ti in range(_TI)
    ]

    pltpu.sync_copy(w_hbm, w_v)
    pltpu.sync_copy(idx_hbm, idx_v)

    def splat_body(j, carry):
        jspl = jnp.full((_L,), j, jnp.int32)
        wexp_v[pl.ds(j * _L, _L)] = plsc.load_gather(w_v, [jspl])
        cexp_v[pl.ds(j * _L, _L)] = plsc.load_gather(idx_v, [jspl])
        return carry

    lax.fori_loop(0, _NIN, splat_body, 0, unroll=4)

    zero = jnp.zeros((_L,), jnp.float32)

    def zero_body(i, carry):
        pad_v[pl.ds(i * _L, _L)] = zero
        return carry

    lax.fori_loop(0, _CHUNK * _NPAD // _L, zero_body, 0, unroll=8)

    lane = lax.iota(jnp.int32, _L)
    rowpad_regs = tuple((lane + rg * _L) * _NPAD for rg in range(_RG))

    for d in x_dmas:
        d.wait()

    def pair_body(p, carry):
        rowpad_r = carry
        for half in range(2):
            out_v = out_bufs[half]
            sem_o = o_sems[half]
            row0 = base_row + p * (2 * _CHUNK) + half * _CHUNK
            c0 = half * _CHUNK

            @pl.when(p > 0)
            def _():
                pltpu.make_async_copy(
                    out_v, out_hbm.at[pl.ds(row0, _CHUNK)], sem_o).wait()

            def j_body(k, carry2):
                rowpad_rr = carry2
                for u in range(_UJ):
                    j = k * _UJ + u
                    ti = j // _TI
                    i = j - ti * _TI
                    jrow = ti * _TJW + p
                    wspl = wexp_v[pl.ds(j * _L, _L)]
                    cspl = cexp_v[pl.ds(j * _L, _L)]
                    vals = tuple(
                        x_v[jrow, i, pl.ds(c0 + rg * _L, _L)] * wspl
                        for rg in range(_RG))
                    for rg in range(_RG):
                        plsc.store_scatter(
                            pad_v, [rowpad_rr[rg] + cspl], vals[rg])
                return carry2

            lax.fori_loop(0, _NIN // _UJ, j_body, rowpad_r)

            def compact_body(r, carry3):
                base = r * _NPAD
                for k in range(_NOUT // _L):
                    out_v[r, pl.ds(k * _L, _L)] = \
                        pad_v[pl.ds(base + k * _L, _L)]
                return carry3

            lax.fori_loop(0, _CHUNK, compact_body, 0, unroll=2)

            pltpu.async_copy(
                out_v, out_hbm.at[pl.ds(row0, _CHUNK)], sem_o)
        return rowpad_r

    lax.fori_loop(0, _NCHUNKS // 2, pair_body, rowpad_regs)

    for b in range(2):
        pltpu.make_async_copy(
            out_bufs[b], out_hbm.at[pl.ds(base_row, _CHUNK)],
            o_sems[b]).wait()


def kernel(X_in, weights, input_node_order):
    mesh = plsc.VectorSubcoreMesh(
        core_axis_name="c", subcore_axis_name="s",
        num_cores=_NC, num_subcores=_NS,
    )
    f = pl.kernel(
        _sc_body,
        out_type=jax.ShapeDtypeStruct((_BATCH, _NOUT), jnp.float32),
        mesh=mesh,
        compiler_params=pltpu.CompilerParams(needs_layout_passes=False),
        scratch_types=[
            pltpu.VMEM((_TI * _TJW, _TI, _TC), jnp.float32),
            pltpu.VMEM((_CHUNK * _NPAD,), jnp.float32),
            pltpu.VMEM((_CHUNK, _NOUT), jnp.float32),
            pltpu.VMEM((_CHUNK, _NOUT), jnp.float32),
            pltpu.VMEM((_NIN,), jnp.float32),
            pltpu.VMEM((_NIN,), jnp.int32),
            pltpu.VMEM((_NIN * _L,), jnp.float32),
            pltpu.VMEM((_NIN * _L,), jnp.int32),
            pltpu.SemaphoreType.DMA,
            pltpu.SemaphoreType.DMA,
            pltpu.SemaphoreType.DMA,
        ],
    )
    q = X_in.reshape(_BATCH // _TC, _TC, _TI, _TI).transpose(2, 0, 3, 1)
    q2 = q.reshape(_TI * (_BATCH // _TC), _TI, _TC)
    return f(q2, weights, input_node_order)


# speedup vs baseline: 1.8408x; 1.8408x over previous
import jax
import jax.numpy as jnp
from jax import lax
from jax.experimental import pallas as pl
from jax.experimental.pallas import tpu as pltpu
from jax.experimental.pallas import tpu_sc as plsc

_BATCH = 32768
_NIN = 64
_NOUT = 256
_NC = 2
_NS = 16
_NW = _NC * _NS
_ROWS_PER_W = _BATCH // _NW
_CHUNK = 128
_NCHUNKS = _ROWS_PER_W // _CHUNK
_L = 16
_G = _NIN // _L
_U = 4


def _sc_body(x_hbm, w_hbm, idx_hbm, out_hbm,
             x_v0, x_v1, out_v0, out_v1, w_v, idx_v,
             sem_x0, sem_x1, sem_o0, sem_o1):
    wid = lax.axis_index("s") * _NC + lax.axis_index("c")
    base_row = wid * _ROWS_PER_W

    x_bufs = (x_v0, x_v1)
    out_bufs = (out_v0, out_v1)
    x_sems = (sem_x0, sem_x1)
    o_sems = (sem_o0, sem_o1)

    x_dma0 = pltpu.async_copy(x_hbm.at[pl.ds(base_row, _CHUNK)], x_v0, sem_x0)

    pltpu.sync_copy(w_hbm, w_v)
    pltpu.sync_copy(idx_hbm, idx_v)

    zero = jnp.zeros((_L,), jnp.float32)

    def zero_body(i, carry):
        r = i // (_NOUT // _L)
        k = (i % (_NOUT // _L)) * _L
        for b in range(2):
            out_bufs[b][r, pl.ds(k, _L)] = zero
            out_bufs[b][r + _CHUNK // 2, pl.ds(k, _L)] = zero
        return carry

    lax.fori_loop(0, _CHUNK // 2 * (_NOUT // _L), zero_body, 0,
                  unroll=4)

    w_regs = tuple(w_v[pl.ds(g * _L, _L)] for g in range(_G))
    idx_regs = tuple(idx_v[pl.ds(g * _L, _L)] for g in range(_G))

    x_dmas = [x_dma0, None]
    o_dmas = [None, None]
    for ci in range(_NCHUNKS):
        b = ci % 2
        row0 = base_row + ci * _CHUNK
        if ci + 1 < _NCHUNKS:
            nb = (ci + 1) % 2
            x_dmas[nb] = pltpu.async_copy(
                x_hbm.at[pl.ds(row0 + _CHUNK, _CHUNK)], x_bufs[nb], x_sems[nb])
        x_dmas[b].wait()
        if o_dmas[b] is not None:
            o_dmas[b].wait()

        x_v = x_bufs[b]
        out_v = out_bufs[b]

        def row_body(i, carry):
            w_r, idx_r = carry
            for u in range(_U):
                r = i * _U + u
                vals = tuple(x_v[r, pl.ds(g * _L, _L)] * w_r[g]
                             for g in range(_G))
                rsplat = jnp.full((_L,), r, jnp.int32)
                for g in range(_G):
                    plsc.store_scatter(out_v, [rsplat, idx_r[g]], vals[g])
            return carry

        lax.fori_loop(0, _CHUNK // _U, row_body, (w_regs, idx_regs))

        o_dmas[b] = pltpu.async_copy(
            out_v, out_hbm.at[pl.ds(row0, _CHUNK)], o_sems[b])

    for d in o_dmas:
        if d is not None:
            d.wait()


def kernel(X_in, weights, input_node_order):
    mesh = plsc.VectorSubcoreMesh(
        core_axis_name="c", subcore_axis_name="s",
        num_cores=_NC, num_subcores=_NS,
    )
    f = pl.kernel(
        _sc_body,
        out_type=jax.ShapeDtypeStruct((_BATCH, _NOUT), jnp.float32),
        mesh=mesh,
        compiler_params=pltpu.CompilerParams(needs_layout_passes=False),
        scratch_types=[
            pltpu.VMEM((_CHUNK, _NIN), jnp.float32),
            pltpu.VMEM((_CHUNK, _NIN), jnp.float32),
            pltpu.VMEM((_CHUNK, _NOUT), jnp.float32),
            pltpu.VMEM((_CHUNK, _NOUT), jnp.float32),
            pltpu.VMEM((_NIN,), jnp.float32),
            pltpu.VMEM((_NIN,), jnp.int32),
            pltpu.SemaphoreType.DMA,
            pltpu.SemaphoreType.DMA,
            pltpu.SemaphoreType.DMA,
            pltpu.SemaphoreType.DMA,
        ],
    )
    return f(X_in, weights, input_node_order)
